# baseline (speedup 1.0000x reference)
import functools

import jax
import jax.numpy as jnp
import numpy as np
from jax.experimental import pallas as pl

_LAYER_DIMS = [(192, 8, 32), (256, 8, 32), (256, 8, 32), (256, 8, 24)]


def _vec2skew(v):
    z = jnp.zeros_like(v[..., 0])
    return jnp.stack([
        jnp.stack([z, -v[..., 2], v[..., 1]], axis=-1),
        jnp.stack([v[..., 2], z, -v[..., 0]], axis=-1),
        jnp.stack([-v[..., 1], v[..., 0], z], axis=-1),
    ], axis=-2)


def _skew_to_rmat(v):
    K = _vec2skew(v)
    theta2 = jnp.sum(v * v, axis=-1)[..., None, None]
    theta = jnp.sqrt(theta2 + 1e-20)
    small = theta < 1e-4
    safe_t = jnp.where(small, jnp.ones_like(theta), theta)
    safe_t2 = jnp.where(small, jnp.ones_like(theta2), theta2)
    a = jnp.where(small, 1.0 - theta2 / 6.0, jnp.sin(theta) / safe_t)
    b = jnp.where(small, 0.5 - theta2 / 24.0, (1.0 - jnp.cos(theta)) / safe_t2)
    I = jnp.eye(3, dtype=v.dtype)
    return I + a * K + b * (K @ K)


def _sqrt_pos(x):
    return jnp.where(x > 0, jnp.sqrt(jnp.maximum(x, 1e-20)), jnp.zeros_like(x))


def _matrix_to_quaternion(matrix):
    m00 = matrix[..., 0, 0]; m01 = matrix[..., 0, 1]; m02 = matrix[..., 0, 2]
    m10 = matrix[..., 1, 0]; m11 = matrix[..., 1, 1]; m12 = matrix[..., 1, 2]
    m20 = matrix[..., 2, 0]; m21 = matrix[..., 2, 1]; m22 = matrix[..., 2, 2]
    q_abs = _sqrt_pos(jnp.stack([
        1.0 + m00 + m11 + m22, 1.0 + m00 - m11 - m22,
        1.0 - m00 + m11 - m22, 1.0 - m00 - m11 + m22], axis=-1))
    quat_by_rijk = jnp.stack([
        jnp.stack([q_abs[..., 0] ** 2, m21 - m12, m02 - m20, m10 - m01], axis=-1),
        jnp.stack([m21 - m12, q_abs[..., 1] ** 2, m10 + m01, m02 + m20], axis=-1),
        jnp.stack([m02 - m20, m10 + m01, q_abs[..., 2] ** 2, m12 + m21], axis=-1),
        jnp.stack([m10 - m01, m20 + m02, m21 + m12, q_abs[..., 3] ** 2], axis=-1),
    ], axis=-2)
    quat_candidates = quat_by_rijk / (2.0 * jnp.maximum(q_abs[..., None], 0.1))
    idx = jnp.argmax(q_abs, axis=-1)
    one_hot = jax.nn.one_hot(idx, 4, dtype=matrix.dtype)
    return jnp.sum(quat_candidates * one_hot[..., :, None], axis=-2)


def _transformer_conv(x, src, dst, layer, H, D):
    n = x.shape[0]
    q = (x @ layer['Wq']).reshape(n, H, D)
    k = (x @ layer['Wk']).reshape(n, H, D)
    v = (x @ layer['Wv']).reshape(n, H, D)
    logits = jnp.sum(q[dst] * k[src], axis=-1) / np.sqrt(D)
    m = jax.ops.segment_max(logits, dst, num_segments=n)
    m = jnp.where(jnp.isfinite(m), m, 0.0)
    ex = jnp.exp(logits - m[dst])
    den = jax.ops.segment_sum(ex, dst, num_segments=n)
    alpha = ex / (den[dst] + 1e-16)
    agg = jax.ops.segment_sum(alpha[..., None] * v[src], dst, num_segments=n)
    return agg.reshape(n, H * D) + x @ layer['Wskip'], alpha


def _identity_pallas(x):
    def body(x_ref, o_ref):
        o_ref[...] = x_ref[...]
    return pl.pallas_call(
        body, out_shape=jax.ShapeDtypeStruct(x.shape, x.dtype))(x)


def kernel(xy_pos, time, pcd, edge_index, batch, params):
    h = jax.nn.relu(pcd @ params['pn_W1'] + params['pn_b1'])
    h = jax.nn.relu(h @ params['pn_W2'] + params['pn_b2'])
    h = h @ params['pn_W3'] + params['pn_b3']
    pcd_feats = jnp.max(h, axis=1)
    time_feats = params['time_emb'][time]
    pf = jax.nn.gelu(xy_pos @ params['pos_W1'] + params['pos_b1'], approximate=False)
    pos_feats = pf @ params['pos_W2'] + params['pos_b2']
    combined = jnp.concatenate([pcd_feats, pos_feats, time_feats], axis=-1)
    c = jax.nn.leaky_relu(combined @ params['mlp_W1'] + params['mlp_b1'], negative_slope=0.2)
    combined = jax.nn.leaky_relu(c @ params['mlp_W2'] + params['mlp_b2'], negative_slope=0.2)

    src = edge_index[0]; dst = edge_index[1]
    x = combined
    alpha = None
    for i, (din, H, D) in enumerate(_LAYER_DIMS):
        x, alpha = _transformer_conv(x, src, dst, params['gnn'][i], H, D)
        if i < len(_LAYER_DIMS) - 1:
            x = jax.nn.relu(x)
    feats = x

    ht = jax.nn.gelu((feats + combined) @ params['t_W1'] + params['t_b1'], approximate=False)
    t_pred = ht @ params['t_W2'] + params['t_b2']
    hr = jax.nn.gelu((feats + combined) @ params['r_W1'] + params['r_b1'], approximate=False)
    r_vec = hr @ params['r_W2'] + params['r_b2']
    quat = _matrix_to_quaternion(_skew_to_rmat(r_vec))
    quat = quat / jnp.maximum(jnp.linalg.norm(quat, axis=-1, keepdims=True), 1e-12)
    out = _identity_pallas(jnp.hstack([quat, t_pred]))
    return out, alpha


# speedup vs baseline: 5.5497x; 5.5497x over previous
import functools

import jax
import jax.numpy as jnp
import numpy as np
from jax import lax
from jax.experimental import pallas as pl
from jax.experimental.pallas import tpu as pltpu
from jax.experimental.pallas import tpu_sc as plsc

_N = 10000
_E = 320000
_NP = 10240
_NP4 = _NP * 4
_CB = 128
_NCHUNK = 160
_EPT = _NCHUNK * _CB
_EPAD = 16 * _EPT
_LAYER_DIMS = [(192, 8, 32), (256, 8, 32), (256, 8, 32), (256, 8, 24)]

_NEG = -1.0e30


def _f16(v):
    return jnp.full((16,), v, jnp.int32)


@functools.lru_cache(maxsize=None)
def _edge_kernel(D, want_alpha):
    Wc = 2 * D
    _NP2 = _NP * 2
    inv_sqrt_d = float(1.0 / np.sqrt(D))
    mesh = plsc.VectorSubcoreMesh(core_axis_name="c", subcore_axis_name="s")

    out_type = [
        jax.ShapeDtypeStruct((2, 2, _NP, Wc), jnp.float32),
        jax.ShapeDtypeStruct((2, 16, _NP2), jnp.float32),
        jax.ShapeDtypeStruct((2, _NP2), jnp.float32),
        jax.ShapeDtypeStruct((2, 16, _NCHUNK, 2, _CB), jnp.float32),
        jax.ShapeDtypeStruct((2, _NP2), jnp.float32),
    ]
    if want_alpha:
        out_type.append(
            jax.ShapeDtypeStruct((2, 2, 16, _NCHUNK, 2, _CB), jnp.float32))

    scratch_types = [
        pltpu.VMEM((_CB,), jnp.int32),
        pltpu.VMEM((_CB,), jnp.int32),
        pltpu.VMEM((_NP2,), jnp.float32),
        pltpu.VMEM((_CB, Wc), jnp.float32),
        pltpu.VMEM((_CB, Wc), jnp.float32),
        pltpu.VMEM((_CB, 8), jnp.float32),
        pltpu.VMEM((2, _CB), jnp.float32),
        pltpu.VMEM((_CB,), jnp.int32),
        pltpu.VMEM((_CB,), jnp.int32),
        pltpu.VMEM((1, _CB), jnp.int32),
        pltpu.VMEM((2, 1280), jnp.float32),
        pltpu.VMEM((128,), jnp.float32),
        pltpu.VMEM_SHARED((_NP, Wc), jnp.float32),
        pltpu.VMEM_SHARED((_NP, 8), jnp.float32),
        pltpu.SemaphoreType.DMA,
        pltpu.SemaphoreType.DMA,
    ]

    def body(qf, kf, vf, srcf, dstf,
             agg_o, mstage_o, mmerged_o, exl_o, den2_o, *rest):
        if want_alpha:
            alpha_o = rest[0]
            rest = rest[1:]
        (srcb, dstb, m_loc, qbuf, kbuf, exbuf, lbuf, gq, gk, sidx,
         mbuf, dbuf, aggsh, densh, sem1, sem2) = rest
        cc = lax.axis_index("c")
        ss = lax.axis_index("s")
        r0 = ss * (_NP // 16)
        z16 = jnp.zeros((16,), jnp.float32)

        def _round(rnd, rcarry):
            qoff = (cc * 2 + rnd) * _NP

            def _init_m(i, carry):
                m_loc[pl.ds(i * 16, 16)] = jnp.full((16,), _NEG, jnp.float32)
                return carry
            lax.fori_loop(0, _NP2 // 16, _init_m, 0)

            for r in range(64):
                for w in range(Wc // 16):
                    kbuf[r, pl.ds(w * 16, 16)] = z16
            for i in range(_CB * 8 // 16):
                fl = lax.iota(jnp.int32, 16) + i * 16
                plsc.store_scatter(exbuf, [fl // 8, fl % 8], z16)
            for b in range(_NP // 16 // 64):
                pltpu.sync_copy(kbuf.at[pl.ds(0, 64)],
                                aggsh.at[pl.ds(r0 + b * 64, 64)])
                pltpu.sync_copy(exbuf.at[pl.ds(0, 64)],
                                densh.at[pl.ds(r0 + b * 64, 64)])

            def _pass_a(j, carry):
                base = (ss * _NCHUNK + j) * _CB
                pltpu.sync_copy(srcf.at[pl.ds(base, _CB)], srcb)
                pltpu.sync_copy(dstf.at[pl.ds(base, _CB)], dstb)
                for i in range(_CB // 16):
                    sl = pl.ds(i * 16, 16)
                    gq[sl] = dstb[sl] + qoff
                    gk[sl] = srcb[sl] + qoff
                cp1 = pltpu.async_copy(qf.at[gq], qbuf, sem1)
                cp2 = pltpu.async_copy(kf.at[gk], kbuf, sem2)
                cp1.wait()
                cp2.wait()
                for g in range(_CB // 16):
                    sl = pl.ds(g * 16, 16)
                    ev = lax.iota(jnp.int32, 16) + g * 16
                    dv = dstb[sl]
                    eid = base + g * 16 + lax.iota(jnp.int32, 16)
                    valid = eid < _E
                    for h in range(2):
                        acc = jnp.zeros((16,), jnp.float32)
                        for d in range(D):
                            col = _f16(h * D + d)
                            acc = acc + (plsc.load_gather(qbuf, [ev, col]) *
                                         plsc.load_gather(kbuf, [ev, col]))
                        lg = jnp.where(valid, acc * inv_sqrt_d, _NEG)
                        lbuf[h, sl] = lg
                        didx = dv * 2 + h
                        cur = plsc.load_gather(m_loc, [didx])
                        need = lg > cur

                        def _mcond(nd):
                            return plsc.all_reduce_population_count(nd)[0] > 0

                        def _mbody(nd):
                            plsc.store_scatter(m_loc, [didx], lg, mask=nd)
                            c2 = plsc.load_gather(m_loc, [didx])
                            return jnp.logical_and(nd, lg > c2)

                        lax.while_loop(_mcond, _mbody, need)
                pltpu.sync_copy(lbuf, exl_o.at[cc, ss, j])
                return carry
            lax.fori_loop(0, _NCHUNK, _pass_a, 0)

            pltpu.sync_copy(m_loc, mstage_o.at[cc, ss])
            plsc.subcore_barrier()
            ms = ss * 1280
            pltpu.sync_copy(mstage_o.at[cc, 0, pl.ds(ms, 1280)], mbuf.at[0])
            for t in range(1, 16):
                pltpu.sync_copy(mstage_o.at[cc, t, pl.ds(ms, 1280)], mbuf.at[1])

                def _mmax(i, carry):
                    sl = pl.ds(i * 16, 16)
                    mbuf[0, sl] = jnp.maximum(mbuf[0, sl], mbuf[1, sl])
                    return carry
                lax.fori_loop(0, 80, _mmax, 0)
            pltpu.sync_copy(mbuf.at[0], mmerged_o.at[cc, pl.ds(ms, 1280)])
            plsc.subcore_barrier()
            pltpu.sync_copy(mmerged_o.at[cc], m_loc)

            def _pass_b(j, carry):
                base = (ss * _NCHUNK + j) * _CB
                pltpu.sync_copy(exl_o.at[cc, ss, j], lbuf)
                pltpu.sync_copy(srcf.at[pl.ds(base, _CB)], srcb)
                pltpu.sync_copy(dstf.at[pl.ds(base, _CB)], dstb)
                for i in range(_CB // 16):
                    sl = pl.ds(i * 16, 16)
                    gk[sl] = srcb[sl] + qoff
                    sidx[0, sl] = dstb[sl]
                pltpu.async_copy(vf.at[gk], qbuf, sem1).wait()
                for g in range(_CB // 16):
                    sl = pl.ds(g * 16, 16)
                    ev = lax.iota(jnp.int32, 16) + g * 16
                    dv = dstb[sl]
                    eid = base + g * 16 + lax.iota(jnp.int32, 16)
                    valid = eid < _E
                    for h in range(2):
                        lg = lbuf[h, sl]
                        mg = plsc.load_gather(m_loc, [dv * 2 + h])
                        ex = jnp.where(valid, jnp.exp(lg - mg), 0.0)
                        lbuf[h, sl] = ex
                        plsc.store_scatter(exbuf, [ev, _f16(h)], ex)
                        for d in range(D):
                            col = _f16(h * D + d)
                            vv = plsc.load_gather(qbuf, [ev, col])
                            plsc.store_scatter(qbuf, [ev, col], vv * ex)
                if want_alpha:
                    pltpu.sync_copy(lbuf, exl_o.at[cc, ss, j])
                pltpu.sync_copy(qbuf, aggsh.at[sidx.at[0]], add=True)
                pltpu.sync_copy(exbuf, densh.at[sidx.at[0]], add=True)
                return carry
            lax.fori_loop(0, _NCHUNK, _pass_b, 0)
            plsc.subcore_barrier()

            def _pass_d(b, carry):
                r = r0 + b * 64
                pltpu.sync_copy(aggsh.at[pl.ds(r, 64)], qbuf.at[pl.ds(0, 64)])
                pltpu.sync_copy(densh.at[pl.ds(r, 64)], exbuf.at[pl.ds(0, 64)])
                for g in range(4):
                    nv = lax.iota(jnp.int32, 16) + g * 16
                    for h in range(2):
                        dv = plsc.load_gather(exbuf, [nv, _f16(h)])
                        plsc.store_scatter(dbuf, [nv * 2 + h], dv)
                        rec = 1.0 / (dv + 1e-16)
                        for d in range(D):
                            col = _f16(h * D + d)
                            av = plsc.load_gather(qbuf, [nv, col])
                            plsc.store_scatter(qbuf, [nv, col], av * rec)
                pltpu.sync_copy(qbuf.at[pl.ds(0, 64)],
                                agg_o.at[cc, rnd, pl.ds(r, 64)])
                pltpu.sync_copy(dbuf, den2_o.at[cc, pl.ds(r * 2, 128)])
                return carry
            lax.fori_loop(0, _NP // 16 // 64, _pass_d, 0)

            if want_alpha:
                plsc.subcore_barrier()
                pltpu.sync_copy(den2_o.at[cc], m_loc)

                def _pass_c(j, carry):
                    base = (ss * _NCHUNK + j) * _CB
                    pltpu.sync_copy(exl_o.at[cc, ss, j], lbuf)
                    pltpu.sync_copy(dstf.at[pl.ds(base, _CB)], dstb)
                    for g in range(_CB // 16):
                        sl = pl.ds(g * 16, 16)
                        dv = dstb[sl]
                        for h in range(2):
                            ex = lbuf[h, sl]
                            dg = plsc.load_gather(m_loc, [dv * 2 + h])
                            lbuf[h, sl] = ex / (dg + 1e-16)
                    pltpu.sync_copy(lbuf, alpha_o.at[cc, rnd, ss, j])
                    return carry
                lax.fori_loop(0, _NCHUNK, _pass_c, 0)
            plsc.subcore_barrier()
            return rcarry
        lax.fori_loop(0, 2, _round, 0)

    return pl.kernel(body, out_type=tuple(out_type), mesh=mesh,
                     scratch_types=tuple(scratch_types),
                     compiler_params=pltpu.CompilerParams(
                         needs_layout_passes=False, use_tc_tiling_on_sc=False))


def _gnn_layer(x, srcp, dstp, layer, D, want_alpha):
    Wc = 2 * D
    q = x @ layer['Wq']
    k = x @ layer['Wk']
    v = x @ layer['Wv']

    def _split(t):
        return t.reshape(_NP, 4, Wc).transpose(1, 0, 2).reshape(4 * _NP, Wc)

    outs = _edge_kernel(D, want_alpha)(_split(q), _split(k), _split(v), srcp, dstp)
    agg = outs[0]
    agg_full = jnp.concatenate([agg[0, 0], agg[0, 1], agg[1, 0], agg[1, 1]], axis=-1)
    x_out = agg_full + x @ layer['Wskip']
    alpha = None
    if want_alpha:
        ac = outs[5]
        alpha = ac.transpose(2, 3, 5, 0, 1, 4).reshape(_EPAD, 8)[:_E]
    return x_out, alpha


def _vec2skew(v):
    z = jnp.zeros_like(v[..., 0])
    return jnp.stack([
        jnp.stack([z, -v[..., 2], v[..., 1]], axis=-1),
        jnp.stack([v[..., 2], z, -v[..., 0]], axis=-1),
        jnp.stack([-v[..., 1], v[..., 0], z], axis=-1),
    ], axis=-2)


def _skew_to_rmat(v):
    K = _vec2skew(v)
    theta2 = jnp.sum(v * v, axis=-1)[..., None, None]
    theta = jnp.sqrt(theta2 + 1e-20)
    small = theta < 1e-4
    safe_t = jnp.where(small, jnp.ones_like(theta), theta)
    safe_t2 = jnp.where(small, jnp.ones_like(theta2), theta2)
    a = jnp.where(small, 1.0 - theta2 / 6.0, jnp.sin(theta) / safe_t)
    b = jnp.where(small, 0.5 - theta2 / 24.0, (1.0 - jnp.cos(theta)) / safe_t2)
    I = jnp.eye(3, dtype=v.dtype)
    return I + a * K + b * (K @ K)


def _sqrt_pos(x):
    return jnp.where(x > 0, jnp.sqrt(jnp.maximum(x, 1e-20)), jnp.zeros_like(x))


def _matrix_to_quaternion(matrix):
    m00 = matrix[..., 0, 0]; m01 = matrix[..., 0, 1]; m02 = matrix[..., 0, 2]
    m10 = matrix[..., 1, 0]; m11 = matrix[..., 1, 1]; m12 = matrix[..., 1, 2]
    m20 = matrix[..., 2, 0]; m21 = matrix[..., 2, 1]; m22 = matrix[..., 2, 2]
    q_abs = _sqrt_pos(jnp.stack([
        1.0 + m00 + m11 + m22, 1.0 + m00 - m11 - m22,
        1.0 - m00 + m11 - m22, 1.0 - m00 - m11 + m22], axis=-1))
    quat_by_rijk = jnp.stack([
        jnp.stack([q_abs[..., 0] ** 2, m21 - m12, m02 - m20, m10 - m01], axis=-1),
        jnp.stack([m21 - m12, q_abs[..., 1] ** 2, m10 + m01, m02 + m20], axis=-1),
        jnp.stack([m02 - m20, m10 + m01, q_abs[..., 2] ** 2, m12 + m21], axis=-1),
        jnp.stack([m10 - m01, m20 + m02, m21 + m12, q_abs[..., 3] ** 2], axis=-1),
    ], axis=-2)
    quat_candidates = quat_by_rijk / (2.0 * jnp.maximum(q_abs[..., None], 0.1))
    idx = jnp.argmax(q_abs, axis=-1)
    one_hot = jax.nn.one_hot(idx, 4, dtype=matrix.dtype)
    return jnp.sum(quat_candidates * one_hot[..., :, None], axis=-2)


def kernel(xy_pos, time, pcd, edge_index, batch, params):
    h = jax.nn.relu(pcd @ params['pn_W1'] + params['pn_b1'])
    h = jax.nn.relu(h @ params['pn_W2'] + params['pn_b2'])
    h = h @ params['pn_W3'] + params['pn_b3']
    pcd_feats = jnp.max(h, axis=1)
    time_feats = params['time_emb'][time]
    pf = jax.nn.gelu(xy_pos @ params['pos_W1'] + params['pos_b1'], approximate=False)
    pos_feats = pf @ params['pos_W2'] + params['pos_b2']
    combined = jnp.concatenate([pcd_feats, pos_feats, time_feats], axis=-1)
    c = jax.nn.leaky_relu(combined @ params['mlp_W1'] + params['mlp_b1'], negative_slope=0.2)
    combined = jax.nn.leaky_relu(c @ params['mlp_W2'] + params['mlp_b2'], negative_slope=0.2)

    src = edge_index[0].astype(jnp.int32)
    dst = edge_index[1].astype(jnp.int32)
    srcp = jnp.concatenate([src, jnp.zeros((_EPAD - _E,), jnp.int32)])
    dstp = jnp.concatenate([dst, jnp.zeros((_EPAD - _E,), jnp.int32)])

    x = jnp.pad(combined, ((0, _NP - _N), (0, 0)))
    alpha = None
    for i, (din, H, D) in enumerate(_LAYER_DIMS):
        last = i == len(_LAYER_DIMS) - 1
        x, alpha = _gnn_layer(x, srcp, dstp, params['gnn'][i], D, last)
        if not last:
            x = jax.nn.relu(x)
    feats = x[:_N]

    ht = jax.nn.gelu((feats + combined) @ params['t_W1'] + params['t_b1'], approximate=False)
    t_pred = ht @ params['t_W2'] + params['t_b2']
    hr = jax.nn.gelu((feats + combined) @ params['r_W1'] + params['r_b1'], approximate=False)
    r_vec = hr @ params['r_W2'] + params['r_b2']
    quat = _matrix_to_quaternion(_skew_to_rmat(r_vec))
    quat = quat / jnp.maximum(jnp.linalg.norm(quat, axis=-1, keepdims=True), 1e-12)
    return jnp.hstack([quat, t_pred]), alpha
